# baseline (device time: 72278 ns/iter reference)
import jax
import jax.numpy as jnp
from jax import lax
from jax.experimental import pallas as pl
from jax.experimental.pallas import tpu as pltpu

B = 2
S = 1024
S_HALF = 512
K = 1024
N = 2048

CS = 128
NCB = S_HALF // CS
NCHUNK = B * NCB


def kernel(O, Wo):
    O2 = O.reshape(B, S, K)

    def body(
        o_hbm, w_hbm, out_hbm,
        w_f32, w_bf, o_pk, o_my, send_buf, recv_buf, acc,
        w_sems, op_sems, om_sems, st_sems, send_sems, recv_sems,
    ):
        my_x = lax.axis_index("x")
        my_y = lax.axis_index("y")
        my_z = lax.axis_index("z")
        peer = (1 - my_x, my_y, my_z)

        barrier_sem = pltpu.get_barrier_semaphore()
        pl.semaphore_signal(
            barrier_sem, inc=1, device_id=peer,
            device_id_type=pl.DeviceIdType.MESH,
        )

        peer_start = (1 - my_x) * S_HALF
        my_start = my_x * S_HALF

        w_cps = []
        for h in range(2):
            cp = pltpu.make_async_copy(
                w_hbm.at[pl.ds(h * (K // 2), K // 2), :],
                w_f32.at[pl.ds(h * (K // 2), K // 2), :],
                w_sems.at[h],
            )
            cp.start()
            w_cps.append(cp)
        o_cps = []
        for idx in range(NCHUNK):
            b, c = divmod(idx, NCB)
            cp = pltpu.make_async_copy(
                o_hbm.at[b, pl.ds(peer_start + c * CS, CS), :],
                o_pk.at[idx],
                op_sems.at[idx],
            )
            cp.start()
            o_cps.append(cp)
        m_cps = []
        for b in range(B):
            cp = pltpu.make_async_copy(
                o_hbm.at[b, pl.ds(my_start, S_HALF), :],
                o_my.at[b],
                om_sems.at[b],
            )
            cp.start()
            m_cps.append(cp)

        for h in range(2):
            w_cps[h].wait()
            sl = pl.ds(h * (K // 2), K // 2)
            w_bf[sl, :] = w_f32[sl, :].astype(jnp.bfloat16)
        w = w_bf[...]

        pl.semaphore_wait(barrier_sem, 1)

        rdmas = []
        for idx in range(NCHUNK):
            o_cps[idx].wait()
            a = o_pk[idx].astype(jnp.bfloat16)
            send_buf[idx, :, :] = jnp.dot(
                a, w, preferred_element_type=jnp.float32
            ).astype(jnp.bfloat16)
            rdma = pltpu.make_async_remote_copy(
                src_ref=send_buf.at[idx],
                dst_ref=recv_buf.at[idx],
                send_sem=send_sems.at[idx],
                recv_sem=recv_sems.at[idx],
                device_id=peer,
                device_id_type=pl.DeviceIdType.MESH,
            )
            rdma.start()
            rdmas.append(rdma)

        for b in range(B):
            m_cps[b].wait()
        for idx in range(NCHUNK):
            b, c = divmod(idx, NCB)
            a = o_my[b, pl.ds(c * CS, CS), :].astype(jnp.bfloat16)
            acc[idx, :, :] = jnp.dot(a, w, preferred_element_type=jnp.float32)

        st_cps = []
        for idx, rdma in enumerate(rdmas):
            b, c = divmod(idx, NCB)
            rdma.wait_send()
            rdma.wait_recv()
            acc[idx, :, :] = acc[idx, :, :] + recv_buf[idx].astype(jnp.float32)
            cp = pltpu.make_async_copy(
                acc.at[idx],
                out_hbm.at[b, pl.ds(c * CS, CS), :],
                st_sems.at[idx],
            )
            cp.start()
            st_cps.append(cp)
        for cp in st_cps:
            cp.wait()

    return pl.pallas_call(
        body,
        out_shape=jax.ShapeDtypeStruct((B, S_HALF, N), jnp.float32),
        in_specs=[
            pl.BlockSpec(memory_space=pltpu.MemorySpace.HBM),
            pl.BlockSpec(memory_space=pltpu.MemorySpace.HBM),
        ],
        out_specs=pl.BlockSpec(memory_space=pltpu.MemorySpace.HBM),
        scratch_shapes=[
            pltpu.VMEM((K, N), jnp.float32),
            pltpu.VMEM((K, N), jnp.bfloat16),
            pltpu.VMEM((NCHUNK, CS, K), jnp.float32),
            pltpu.VMEM((B, S_HALF, K), jnp.float32),
            pltpu.VMEM((NCHUNK, CS, N), jnp.bfloat16),
            pltpu.VMEM((NCHUNK, CS, N), jnp.bfloat16),
            pltpu.VMEM((NCHUNK, CS, N), jnp.float32),
            pltpu.SemaphoreType.DMA((2,)),
            pltpu.SemaphoreType.DMA((NCHUNK,)),
            pltpu.SemaphoreType.DMA((B,)),
            pltpu.SemaphoreType.DMA((NCHUNK,)),
            pltpu.SemaphoreType.DMA((NCHUNK,)),
            pltpu.SemaphoreType.DMA((NCHUNK,)),
        ],
        compiler_params=pltpu.CompilerParams(
            collective_id=0, vmem_limit_bytes=100 * 1024 * 1024
        ),
    )(O2, Wo)


# device time: 65608 ns/iter; 1.1017x vs baseline; 1.1017x over previous
import jax
import jax.numpy as jnp
from jax import lax
from jax.experimental import pallas as pl
from jax.experimental.pallas import tpu as pltpu

B = 2
S = 1024
S_HALF = 512
K = 1024
N = 2048

CS = 128
NCB = S_HALF // CS
NCHUNK = B * NCB

_DN = (((0,), (0,)), ((), ()))


def kernel(O, Wo):
    O2T = O.reshape(B, S, K).transpose(0, 2, 1)

    def body(o_ref, w_ref, out_ref, send_buf, recv_buf, send_sems, recv_sems):
        my_x = lax.axis_index("x")
        my_y = lax.axis_index("y")
        my_z = lax.axis_index("z")
        peer = (1 - my_x, my_y, my_z)

        barrier_sem = pltpu.get_barrier_semaphore()
        pl.semaphore_signal(
            barrier_sem, inc=1, device_id=peer,
            device_id_type=pl.DeviceIdType.MESH,
        )
        pl.semaphore_wait(barrier_sem, 1)

        w = w_ref[...].astype(jnp.bfloat16)
        peer_start = (1 - my_x) * S_HALF
        my_start = my_x * S_HALF

        rdmas = []
        for idx in range(NCHUNK):
            b, c = divmod(idx, NCB)
            lhsT = o_ref[b, :, pl.ds(peer_start + c * CS, CS)].astype(
                jnp.bfloat16
            )
            send_buf[idx, :, :] = lax.dot_general(
                lhsT, w, _DN, preferred_element_type=jnp.float32
            ).astype(jnp.bfloat16)
            rdma = pltpu.make_async_remote_copy(
                src_ref=send_buf.at[idx],
                dst_ref=recv_buf.at[idx],
                send_sem=send_sems.at[idx],
                recv_sem=recv_sems.at[idx],
                device_id=peer,
                device_id_type=pl.DeviceIdType.MESH,
            )
            rdma.start()
            rdmas.append(rdma)

        for b in range(B):
            lhsT = o_ref[b, :, pl.ds(my_start, S_HALF)].astype(jnp.bfloat16)
            out_ref[b, :, :] = lax.dot_general(
                lhsT, w, _DN, preferred_element_type=jnp.float32
            )

        for idx, rdma in enumerate(rdmas):
            b, c = divmod(idx, NCB)
            rdma.wait_send()
            rdma.wait_recv()
            sl = pl.ds(c * CS, CS)
            out_ref[b, sl, :] = out_ref[b, sl, :] + recv_buf[idx].astype(
                jnp.float32
            )

    return pl.pallas_call(
        body,
        out_shape=jax.ShapeDtypeStruct((B, S_HALF, N), jnp.float32),
        in_specs=[
            pl.BlockSpec(memory_space=pltpu.VMEM),
            pl.BlockSpec(memory_space=pltpu.VMEM),
        ],
        out_specs=pl.BlockSpec(memory_space=pltpu.VMEM),
        scratch_shapes=[
            pltpu.VMEM((NCHUNK, CS, N), jnp.bfloat16),
            pltpu.VMEM((NCHUNK, CS, N), jnp.bfloat16),
            pltpu.SemaphoreType.DMA((NCHUNK,)),
            pltpu.SemaphoreType.DMA((NCHUNK,)),
        ],
        compiler_params=pltpu.CompilerParams(
            collective_id=0, vmem_limit_bytes=100 * 1024 * 1024
        ),
    )(O2T, Wo)
